# baseline (speedup 1.0000x reference)
import functools

import jax
import jax.numpy as jnp
from jax import lax
from jax.experimental import pallas as pl
from jax.experimental.pallas import tpu as pltpu
from jax.experimental.pallas import tpu_sc as plsc

NC = 2
NS = 16
NW = NC * NS
LANES = 16
EMBED = 128
CHUNKS = (64, 128, 128, 128, 64)


def _sc_partials(x_flat, score_flat, table, bpw):
    n_vregs = EMBED // LANES
    nch = len(CHUNKS)
    offs = [sum(CHUNKS[:g]) for g in range(nch)]
    mesh = plsc.VectorSubcoreMesh(
        core_axis_name="c", subcore_axis_name="s",
        num_cores=NC, num_subcores=NS)

    @functools.partial(
        pl.kernel,
        out_type=jax.ShapeDtypeStruct((NW, EMBED), jnp.float32),
        mesh=mesh,
        scratch_types=(
            [pltpu.VMEM((bpw,), jnp.int32),
             pltpu.VMEM((bpw,), jnp.float32)]
            + [pltpu.VMEM((c, EMBED), jnp.float32)
               for c in CHUNKS]
            + [pltpu.VMEM((EMBED,), jnp.float32)]
            + [pltpu.SemaphoreType.DMA for _ in CHUNKS]
        ),
    )
    def body(x_hbm, s_hbm, table_hbm, out_hbm, idx_v, score_v, *scratch):
        bufs = scratch[:nch]
        part_v = scratch[nch]
        sems = scratch[nch + 1:]
        wid = lax.axis_index("s") * NC + lax.axis_index("c")
        pltpu.sync_copy(x_hbm.at[pl.ds(wid * bpw, bpw)], idx_v)

        copies = [None] * nch

        def fire(g):
            copies[g] = pltpu.async_copy(
                table_hbm.at[idx_v.at[pl.ds(offs[g], CHUNKS[g])]],
                bufs[g], sems[g])

        fire(0)
        fire(1)
        pltpu.sync_copy(s_hbm.at[pl.ds(wid * bpw, bpw)], score_v)
        acc = tuple(jnp.zeros((LANES,), jnp.float32) for _ in range(n_vregs))
        for g in range(nch):
            if g + 2 < nch:
                fire(g + 2)
            copies[g].wait()
            rows_v = bufs[g]

            def grp_body(k, a, g=g, rows_v=rows_v):
                svec = score_v[pl.ds(offs[g] + k * LANES, LANES)]
                for l in range(LANES):
                    sv = jnp.full((LANES,), svec[l])
                    r = k * LANES + l
                    a = tuple(
                        a[j] + rows_v[r, pl.ds(j * LANES, LANES)] * sv
                        for j in range(n_vregs))
                return a

            acc = plsc.parallel_loop(
                0, CHUNKS[g] // LANES, unroll=1, carry=acc)(grp_body)

        for j in range(n_vregs):
            part_v[pl.ds(j * LANES, LANES)] = acc[j]
        pltpu.sync_copy(part_v, out_hbm.at[wid])

    return body(x_flat, score_flat, table)


def _tc_finish(partials, W, b2, batch):
    def body(p_ref, w_ref, b_ref, o_ref):
        m = jnp.sum(p_ref[...], axis=0, keepdims=True) * (1.0 / batch)
        val = jnp.sum(m * w_ref[...]).reshape(1, 1) + b_ref[...]
        o_ref[...] = jax.nn.sigmoid(val)

    return pl.pallas_call(
        body,
        out_shape=jax.ShapeDtypeStruct((1, 1), jnp.float32),
    )(partials, W, b2)


def kernel(x, score, table, W, b):
    batch = x.shape[0]
    bpw = batch // NW
    x_flat = x.astype(jnp.int32).reshape(batch)
    score_flat = score.reshape(batch)
    partials = _sc_partials(x_flat, score_flat, table, bpw)
    out = _tc_finish(partials, W, b.reshape(1, 1), batch)
    return out.reshape(1)


# speedup vs baseline: 1.0261x; 1.0261x over previous
import functools

import jax
import jax.numpy as jnp
from jax import lax
from jax.experimental import pallas as pl
from jax.experimental.pallas import tpu as pltpu
from jax.experimental.pallas import tpu_sc as plsc

NC = 2
NS = 16
NW = NC * NS
LANES = 16
EMBED = 128
CHUNK = 128
NBUF = 2


def _sc_partials(x2, score_flat, table, bpw, cpw):
    n_vregs = EMBED // LANES
    mesh = plsc.VectorSubcoreMesh(
        core_axis_name="c", subcore_axis_name="s",
        num_cores=NC, num_subcores=NS)

    @functools.partial(
        pl.kernel,
        out_type=jax.ShapeDtypeStruct((NW, EMBED), jnp.float32),
        mesh=mesh,
        scratch_types=(
            [pltpu.VMEM((cpw, CHUNK), jnp.int32),
             pltpu.VMEM((bpw,), jnp.float32)]
            + [pltpu.VMEM((CHUNK, EMBED), jnp.float32)
               for _ in range(NBUF)]
            + [pltpu.VMEM((EMBED,), jnp.float32)]
            + [pltpu.SemaphoreType.DMA for _ in range(NBUF)]
        ),
    )
    def body(x_hbm, s_hbm, table_hbm, out_hbm, idx_v, score_v, *scratch):
        bufs = scratch[:NBUF]
        part_v = scratch[NBUF]
        sems = scratch[NBUF + 1:]
        wid = lax.axis_index("s") * NC + lax.axis_index("c")
        pltpu.sync_copy(x_hbm.at[pl.ds(wid * cpw, cpw)], idx_v)

        copies = [None] * NBUF

        def fire(g):
            copies[g % NBUF] = pltpu.async_copy(
                table_hbm.at[idx_v.at[g]], bufs[g % NBUF], sems[g % NBUF])

        for g in range(min(NBUF - 1, cpw)):
            fire(g)
        pltpu.sync_copy(s_hbm.at[pl.ds(wid * bpw, bpw)], score_v)
        acc = tuple(jnp.zeros((LANES,), jnp.float32) for _ in range(n_vregs))
        for g in range(cpw):
            if g + NBUF - 1 < cpw:
                fire(g + NBUF - 1)
            copies[g % NBUF].wait()
            rows_v = bufs[g % NBUF]

            def grp_body(k, a, g=g, rows_v=rows_v):
                svec = score_v[pl.ds(g * CHUNK + k * LANES, LANES)]
                for l in range(LANES):
                    sv = jnp.full((LANES,), svec[l])
                    r = k * LANES + l
                    a = tuple(
                        a[j] + rows_v[r, pl.ds(j * LANES, LANES)] * sv
                        for j in range(n_vregs))
                return a

            acc = plsc.parallel_loop(
                0, CHUNK // LANES, unroll=1, carry=acc)(grp_body)

        for j in range(n_vregs):
            part_v[pl.ds(j * LANES, LANES)] = acc[j]
        pltpu.sync_copy(part_v, out_hbm.at[wid])

    return body(x2, score_flat, table)


def _tc_finish(partials, W, b2, batch):
    def body(p_ref, w_ref, b_ref, o_ref):
        m = jnp.sum(p_ref[...], axis=0, keepdims=True) * (1.0 / batch)
        val = jnp.sum(m * w_ref[...]).reshape(1, 1) + b_ref[...]
        o_ref[...] = jax.nn.sigmoid(val)

    return pl.pallas_call(
        body,
        out_shape=jax.ShapeDtypeStruct((1, 1), jnp.float32),
    )(partials, W, b2)


def kernel(x, score, table, W, b):
    batch = x.shape[0]
    bpw = batch // NW
    cpw = bpw // CHUNK
    x2 = x.astype(jnp.int32).reshape(NW * cpw, CHUNK)
    score_flat = score.reshape(batch)
    partials = _sc_partials(x2, score_flat, table, bpw, cpw)
    out = _tc_finish(partials, W, b.reshape(1, 1), batch)
    return out.reshape(1)


# speedup vs baseline: 1.0738x; 1.0465x over previous
import functools

import jax
import jax.numpy as jnp
from jax import lax
from jax.experimental import pallas as pl
from jax.experimental.pallas import tpu as pltpu
from jax.experimental.pallas import tpu_sc as plsc

NC = 2
NS = 16
NW = NC * NS
LANES = 16
EMBED = 128
CHUNK = 128
NBUF = 2


def _sc_partials(x2, score_flat, table, bpw, cpw):
    n_vregs = EMBED // LANES
    mesh = plsc.VectorSubcoreMesh(
        core_axis_name="c", subcore_axis_name="s",
        num_cores=NC, num_subcores=NS)

    @functools.partial(
        pl.kernel,
        out_type=jax.ShapeDtypeStruct((NW, EMBED), jnp.float32),
        mesh=mesh,
        scratch_types=(
            [pltpu.VMEM((cpw, CHUNK), jnp.int32),
             pltpu.VMEM((bpw,), jnp.float32)]
            + [pltpu.VMEM((CHUNK, EMBED), jnp.float32)
               for _ in range(NBUF)]
            + [pltpu.VMEM((EMBED,), jnp.float32)]
            + [pltpu.SemaphoreType.DMA for _ in range(NBUF)]
        ),
    )
    def body(x_hbm, s_hbm, table_hbm, out_hbm, idx_v, score_v, *scratch):
        bufs = scratch[:NBUF]
        part_v = scratch[NBUF]
        sems = scratch[NBUF + 1:]
        wid = lax.axis_index("s") * NC + lax.axis_index("c")
        pltpu.sync_copy(x_hbm.at[pl.ds(wid * cpw, cpw)], idx_v)

        def fire(g, b):
            return pltpu.async_copy(
                table_hbm.at[idx_v.at[g]], bufs[b], sems[b])

        fire(0, 0)
        fire(1, 1)
        pltpu.sync_copy(s_hbm.at[pl.ds(wid * bpw, bpw)], score_v)
        acc0 = tuple(jnp.zeros((LANES,), jnp.float32) for _ in range(n_vregs))

        def pair_body(p, acc):
            for half in range(2):
                g = 2 * p + half
                b = half
                pltpu.make_async_copy(
                    table_hbm.at[idx_v.at[g]], bufs[b], sems[b]).wait()
                rows_v = bufs[b]

                def grp_body(k, a, g=g, rows_v=rows_v):
                    svec = score_v[pl.ds(g * CHUNK + k * LANES, LANES)]
                    for l in range(LANES):
                        sv = jnp.full((LANES,), svec[l])
                        r = k * LANES + l
                        a = tuple(
                            a[j] + rows_v[r, pl.ds(j * LANES, LANES)] * sv
                            for j in range(n_vregs))
                    return a

                acc = plsc.parallel_loop(
                    0, CHUNK // LANES, unroll=1, carry=acc)(grp_body)

                @pl.when(g + 2 < cpw)
                def _(g=g, b=b):
                    fire(g + 2, b)
            return acc

        acc = lax.fori_loop(0, cpw // 2, pair_body, acc0)

        for j in range(n_vregs):
            part_v[pl.ds(j * LANES, LANES)] = acc[j]
        pltpu.sync_copy(part_v, out_hbm.at[wid])

    return body(x2, score_flat, table)


def _tc_finish(partials, W, b2, batch):
    def body(p_ref, w_ref, b_ref, o_ref):
        m = jnp.sum(p_ref[...], axis=0, keepdims=True) * (1.0 / batch)
        val = jnp.sum(m * w_ref[...]).reshape(1, 1) + b_ref[...]
        o_ref[...] = jax.nn.sigmoid(val)

    return pl.pallas_call(
        body,
        out_shape=jax.ShapeDtypeStruct((1, 1), jnp.float32),
    )(partials, W, b2)


def kernel(x, score, table, W, b):
    batch = x.shape[0]
    bpw = batch // NW
    cpw = bpw // CHUNK
    x2 = x.astype(jnp.int32).reshape(NW * cpw, CHUNK)
    score_flat = score.reshape(batch)
    partials = _sc_partials(x2, score_flat, table, bpw, cpw)
    out = _tc_finish(partials, W, b.reshape(1, 1), batch)
    return out.reshape(1)
